# Initial kernel scaffold; baseline (speedup 1.0000x reference)
import jax
import jax.numpy as jnp
from jax.experimental import pallas as pl


def kernel(x, e, W):
    raise NotImplementedError("write your pallas kernel here")


# speedup vs baseline: 1.9498x; 1.9498x over previous
import functools

import jax
import jax.numpy as jnp
from jax.experimental import pallas as pl

_N = 131072
_D = 256
_T = 640
_BETA = 0.25
_BN = 512


def _vq_block(at_ref, e_ref, w_ref, q_ref, loss_ref):
    e = e_ref[...]
    w = w_ref[...]
    at = at_ref[0]

    s = jnp.dot(e, w.T, preferred_element_type=jnp.float32)
    w_sq = jnp.sum(w * w, axis=1)
    e_sq = jnp.sum(e * e, axis=1, keepdims=True)
    d = e_sq + w_sq[None, :] - 2.0 * s

    cols = jax.lax.broadcasted_iota(jnp.int32, (1, _T), 1)
    col_seg = ((cols >= 129).astype(jnp.int32)
               + (cols >= 257).astype(jnp.int32)
               + (cols >= 385).astype(jnp.int32)
               + (cols >= 513).astype(jnp.int32)
               + (cols >= 577).astype(jnp.int32))
    invalid = ((cols == 128) | (cols == 256) | (cols == 384)
               | (cols == 512) | (cols == 576))

    seg = jnp.where(at == 5, 0,
          jnp.where(at == 6, 1,
          jnp.where(at == 7, 2,
          jnp.where(at == 119, 4,
          jnp.where(at == 120, 5, 3)))))

    d = jnp.where((col_seg != seg) | invalid, jnp.inf, d)
    idx = jnp.argmin(d, axis=1).astype(jnp.int32)[:, None]
    mind = jnp.min(d, axis=1)

    onehot = (cols == idx).astype(jnp.float32)
    q = jnp.dot(onehot, w, preferred_element_type=jnp.float32)
    q_ref[...] = e + (q - e)

    @pl.when(pl.program_id(0) == 0)
    def _():
        loss_ref[...] = jnp.zeros((1, 1), jnp.float32)
    loss_ref[...] += jnp.sum(mind).reshape(1, 1)


@jax.jit
def _vq(at3, e, W):
    nb = _N // _BN
    q, loss_sum = pl.pallas_call(
        _vq_block,
        grid=(nb,),
        in_specs=[
            pl.BlockSpec((1, _BN, 1), lambda i: (i, 0, 0)),
            pl.BlockSpec((_BN, _D), lambda i: (i, 0)),
            pl.BlockSpec((_T, _D), lambda i: (0, 0)),
        ],
        out_specs=[
            pl.BlockSpec((_BN, _D), lambda i: (i, 0)),
            pl.BlockSpec((1, 1), lambda i: (0, 0)),
        ],
        out_shape=[
            jax.ShapeDtypeStruct((_N, _D), jnp.float32),
            jax.ShapeDtypeStruct((1, 1), jnp.float32),
        ],
    )(at3, e, W)
    return q, loss_sum


def kernel(x, e, W):
    at3 = x[:, 0].astype(jnp.int32).reshape(_N // _BN, _BN, 1)
    q, loss_sum = _vq(at3, e, W)
    cl = loss_sum[0, 0] / (_N * _D)
    gl = cl
    vq_loss = cl + _BETA * gl
    return (q, cl, gl, vq_loss)
